# Initial kernel scaffold; baseline (speedup 1.0000x reference)
import jax
import jax.numpy as jnp
from jax.experimental import pallas as pl


def kernel(logits, rois, appearance_features, raw_sigma, raw_smoothness):
    raise NotImplementedError("write your pallas kernel here")


# speedup vs baseline: 5.5264x; 5.5264x over previous
import functools

import jax
import jax.numpy as jnp
from jax import lax
from jax.experimental import pallas as pl
from jax.experimental.pallas import tpu as pltpu

_N = 2048
_C = 32
_D = 256
_KNN = 32
_RT = 512
_R = 256
_N_ITER = 5


def _masked_dist(rois_all, rois_row, row0, nrows):
    ra = rois_all
    rr = rois_row
    ca = (ra[:, :3] + ra[:, 3:6]) * 0.5
    cr = (rr[:, :3] + rr[:, 3:6]) * 0.5
    sqa = jnp.sum(ca * ca, axis=1)
    sqr = jnp.sum(cr * cr, axis=1)
    valida = jnp.sum(jnp.abs(ra), axis=1) > 0.0
    validr = jnp.sum(jnp.abs(rr), axis=1) > 0.0

    dots = lax.dot_general(cr, ca, (((1,), (1,)), ((), ())),
                           preferred_element_type=jnp.float32)
    dist = sqr[:, None] + sqa[None, :] - 2.0 * dots
    dist = jnp.maximum(dist, 0.0)

    row_ids = row0 + lax.broadcasted_iota(jnp.int32, (nrows, _N), 0)
    col_ids = lax.broadcasted_iota(jnp.int32, (nrows, _N), 1)
    pairmask = validr[:, None] & valida[None, :] & (row_ids != col_ids)
    md = jnp.where(pairmask, dist, jnp.inf)
    return dist, md


def _thresh_kernel(rois_all_ref, rois_row_ref, t_ref):
    i = pl.program_id(1)
    _, md0 = _masked_dist(rois_all_ref[0], rois_row_ref[0], i * _RT, _RT)

    def body(_, carry):
        md, _t = carry
        rowmin = jnp.min(md, axis=1, keepdims=True)
        md = jnp.where(md <= rowmin, jnp.inf, md)
        return md, rowmin

    _, t = lax.fori_loop(0, _KNN, body,
                         (md0, jnp.zeros((_RT, 1), jnp.float32)))
    t_ref[0] = t


def _weights_kernel(rois_all_ref, rois_row_ref, feats_all_ref, feats_row_ref,
                    t_all_ref, t_row_ref, sigma_ref, w_ref):
    i = pl.program_id(1)
    dist, md0 = _masked_dist(rois_all_ref[0], rois_row_ref[0], i * _R, _R)

    t_row = t_row_ref[0]
    t_col = t_all_ref[0].T
    tmax = jnp.maximum(t_row, t_col)
    adj = (md0 <= tmax) & jnp.isfinite(md0)

    sigma = sigma_ref[0, 0]
    gk = jnp.exp(dist * (-0.5 / (sigma * sigma)))

    fa = feats_all_ref[0]
    fr = feats_row_ref[0]
    na = jnp.sqrt(jnp.sum(fa * fa, axis=1, keepdims=True))
    nr = jnp.sqrt(jnp.sum(fr * fr, axis=1, keepdims=True))
    fa = fa / jnp.maximum(na, 1e-6)
    fr = fr / jnp.maximum(nr, 1e-6)
    cos = lax.dot_general(fr, fa, (((1,), (1,)), ((), ())),
                          preferred_element_type=jnp.float32)
    aff = jnp.clip((cos + 1.0) * 0.5, 0.0, 1.0)

    wu = jnp.where(adj, gk * aff, 0.0)
    rs = jnp.sum(wu, axis=1, keepdims=True)
    w_ref[0] = (wu / jnp.maximum(rs, 1e-6)).astype(jnp.bfloat16)


def _crf_kernel(w_ref, logits_ref, smooth_ref, out_ref):
    lg = logits_ref[0]
    wb = w_ref[0]
    smooth = smooth_ref[0, 0]
    ii = lax.broadcasted_iota(jnp.int32, (_C, _C), 0)
    jj = lax.broadcasted_iota(jnp.int32, (_C, _C), 1)
    compat = ((ii - jj) ** 2).astype(jnp.float32)
    compat = compat / jnp.maximum(jnp.max(compat), 1.0)

    q = jax.nn.softmax(lg, axis=-1)

    def body(_, carry):
        q, _refined = carry
        qn = jnp.dot(wb, q.astype(jnp.bfloat16),
                     preferred_element_type=jnp.float32)
        pt = jnp.dot(qn, compat, preferred_element_type=jnp.float32)
        refined = lg - smooth * pt
        q = jax.nn.softmax(refined, axis=-1)
        return q, refined

    _, refined = lax.fori_loop(0, _N_ITER, body, (q, lg))
    out_ref[0] = refined


def kernel(logits, rois, appearance_features, raw_sigma, raw_smoothness):
    B, N, C = logits.shape
    sigma = jnp.maximum(jax.nn.softplus(raw_sigma), 1e-6).astype(jnp.float32)
    smooth = jax.nn.softplus(raw_smoothness).astype(jnp.float32)
    sigma = sigma.reshape(1, 1)
    smooth = smooth.reshape(1, 1)

    t = pl.pallas_call(
        _thresh_kernel,
        grid=(B, N // _RT),
        in_specs=[
            pl.BlockSpec((1, N, 6), lambda b, i: (b, 0, 0)),
            pl.BlockSpec((1, _RT, 6), lambda b, i: (b, i, 0)),
        ],
        out_specs=pl.BlockSpec((1, _RT, 1), lambda b, i: (b, i, 0)),
        out_shape=jax.ShapeDtypeStruct((B, N, 1), jnp.float32),
    )(rois, rois)

    w = pl.pallas_call(
        _weights_kernel,
        grid=(B, N // _R),
        in_specs=[
            pl.BlockSpec((1, N, 6), lambda b, i: (b, 0, 0)),
            pl.BlockSpec((1, _R, 6), lambda b, i: (b, i, 0)),
            pl.BlockSpec((1, N, _D), lambda b, i: (b, 0, 0)),
            pl.BlockSpec((1, _R, _D), lambda b, i: (b, i, 0)),
            pl.BlockSpec((1, N, 1), lambda b, i: (b, 0, 0)),
            pl.BlockSpec((1, _R, 1), lambda b, i: (b, i, 0)),
            pl.BlockSpec((1, 1), lambda b, i: (0, 0)),
        ],
        out_specs=pl.BlockSpec((1, _R, N), lambda b, i: (b, i, 0)),
        out_shape=jax.ShapeDtypeStruct((B, N, N), jnp.bfloat16),
    )(rois, rois, appearance_features, appearance_features, t, t, sigma)

    refined = pl.pallas_call(
        _crf_kernel,
        grid=(B,),
        in_specs=[
            pl.BlockSpec((1, N, N), lambda b: (b, 0, 0)),
            pl.BlockSpec((1, N, C), lambda b: (b, 0, 0)),
            pl.BlockSpec((1, 1), lambda b: (0, 0)),
        ],
        out_specs=pl.BlockSpec((1, N, C), lambda b: (b, 0, 0)),
        out_shape=jax.ShapeDtypeStruct((B, N, C), jnp.float32),
    )(w, logits, smooth)

    return refined


# speedup vs baseline: 11.2103x; 2.0285x over previous
import functools

import jax
import jax.numpy as jnp
from jax import lax
from jax.experimental import pallas as pl
from jax.experimental.pallas import tpu as pltpu
from jax.experimental.pallas import tpu_sc as plsc

_N = 2048
_C = 32
_D = 256
_KNN = 32
_R = 256
_N_ITER = 5
_NSUB = 32
_RPW = None


def _soa_kernel(rois_ref, soa_ref):
    r = rois_ref[0]
    rt = r.T
    cx = (rt[0:1] + rt[3:4]) * 0.5
    cy = (rt[1:2] + rt[4:5]) * 0.5
    cz = (rt[2:3] + rt[5:6]) * 0.5
    sq = (cx * cx + cy * cy) + cz * cz
    absum = (jnp.abs(rt[0:1]) + jnp.abs(rt[1:2]) + jnp.abs(rt[2:3])
             + jnp.abs(rt[3:4]) + jnp.abs(rt[4:5]) + jnp.abs(rt[5:6]))
    validf = (absum > 0.0).astype(jnp.float32)
    sqp = jnp.where(validf > 0.5, sq, jnp.inf)
    pad = jnp.zeros((4, _N), jnp.float32)
    soa_ref[0] = jnp.concatenate([cx, cy, cz, sqp, pad], axis=0)


def _lane_iota():
    return lax.broadcasted_iota(jnp.int32, (16,), 0)


def _bcast_lane(v, lane):
    idx = jnp.full((16,), lane, jnp.int32)
    return v.at[idx].get(mode="promise_in_bounds")


def _rgather(v, idx):
    return v.at[idx].get(mode="promise_in_bounds")


def _bitonic_sort_vregs(vs):
    li = _lane_iota()
    n = len(vs) * 16
    k = 2
    while k <= n:
        s = k // 2
        while s >= 1:
            if s >= 16:
                sv = s // 16
                for p in range(len(vs) // 2):
                    v = ((p & ~(sv - 1)) << 1) | (p & (sv - 1))
                    up = ((16 * v) & k) == 0
                    a, b = vs[v], vs[v + sv]
                    lo = jnp.minimum(a, b)
                    hi = jnp.maximum(a, b)
                    vs[v], vs[v + sv] = (lo, hi) if up else (hi, lo)
            else:
                ls = s.bit_length() - 1
                lk = k.bit_length() - 1
                for v in range(len(vs)):
                    a = vs[v]
                    b = _rgather(a, li ^ s)
                    if k >= 16:
                        if ((16 * v) & k) != 0:
                            tm = (li & s) != 0
                        else:
                            tm = (li & s) == 0
                    else:
                        tm = (((li >> ls) ^ (li >> lk)) & 1) == 0
                    vs[v] = jnp.where(tm, jnp.minimum(a, b), jnp.maximum(a, b))
            s //= 2
        k *= 2
    return vs


def _sc_thresh_body(soa_hbm, t_hbm, soa_v, tbuf):
    nc = 2
    wid = lax.axis_index("s") * nc + lax.axis_index("c")
    segs_per_batch = _N // _RPW
    b = wid // segs_per_batch
    seg = wid % segs_per_batch
    row0 = seg * _RPW

    pltpu.sync_copy(soa_hbm.at[b], soa_v)

    inf16 = jnp.full((16,), jnp.inf, jnp.float32)
    li = _lane_iota()
    nvec = _N // 16

    def row_body(j, acc):
        nrow = row0 + j
        base = (nrow // 16) * 16
        lane = nrow % 16
        xr = _bcast_lane(soa_v[0, pl.ds(base, 16)], lane)
        yr = _bcast_lane(soa_v[1, pl.ds(base, 16)], lane)
        zr = _bcast_lane(soa_v[2, pl.ds(base, 16)], lane)
        sqr = _bcast_lane(soa_v[3, pl.ds(base, 16)], lane)

        def stream_body(jv, ms):
            o = jv * 16
            vx = soa_v[0, pl.ds(o, 16)]
            vy = soa_v[1, pl.ds(o, 16)]
            vz = soa_v[2, pl.ds(o, 16)]
            vs = soa_v[3, pl.ds(o, 16)]
            dot = (vx * xr + vy * yr) + vz * zr
            c = jnp.maximum((sqr - (dot + dot)) + vs, 0.0)
            out = []
            for i in range(8):
                lo = jnp.minimum(ms[i], c)
                c = jnp.maximum(ms[i], c)
                out.append(lo)
            return tuple(out)

        ms = lax.fori_loop(0, nvec, stream_body, (inf16,) * 8)

        vs_ = _bitonic_sort_vregs(list(ms))
        t = _bcast_lane(vs_[2], 0)

        acc = jnp.where(li == (j & 15), t, acc)
        tbuf[pl.ds((j // 16) * 16, 16)] = acc
        return acc

    lax.fori_loop(0, _RPW, row_body, jnp.zeros((16,), jnp.float32))
    pltpu.sync_copy(tbuf, t_hbm.at[b, pl.ds(row0, _RPW)])


def _weights_kernel(rois_all_ref, rois_row_ref, feats_all_ref, feats_row_ref,
                    t_all_ref, t_row_ref, sigma_ref, w_ref):
    i = pl.program_id(1)
    ra = rois_all_ref[0]
    rr = rois_row_ref[0]

    cxr = (rr[:, 0:1] + rr[:, 3:4]) * 0.5
    cyr = (rr[:, 1:2] + rr[:, 4:5]) * 0.5
    czr = (rr[:, 2:3] + rr[:, 5:6]) * 0.5
    sq_r = (cxr * cxr + cyr * cyr) + czr * czr
    absum_r = (jnp.abs(rr[:, 0:1]) + jnp.abs(rr[:, 1:2]) + jnp.abs(rr[:, 2:3])
               + jnp.abs(rr[:, 3:4]) + jnp.abs(rr[:, 4:5]) + jnp.abs(rr[:, 5:6]))
    sqp_r = jnp.where(absum_r > 0.0, sq_r, jnp.inf)

    rat = ra.T
    cxa = (rat[0:1] + rat[3:4]) * 0.5
    cya = (rat[1:2] + rat[4:5]) * 0.5
    cza = (rat[2:3] + rat[5:6]) * 0.5
    sq_a = (cxa * cxa + cya * cya) + cza * cza
    absum_a = (jnp.abs(rat[0:1]) + jnp.abs(rat[1:2]) + jnp.abs(rat[2:3])
               + jnp.abs(rat[3:4]) + jnp.abs(rat[4:5]) + jnp.abs(rat[5:6]))
    validf_a = (absum_a > 0.0).astype(jnp.float32)
    sqp_a = jnp.where(validf_a > 0.5, sq_a, jnp.inf)

    dot = (cxa * cxr + cya * cyr) + cza * czr
    d = jnp.maximum((sqp_r - (dot + dot)) + sqp_a, 0.0)

    row_ids = i * _R + lax.broadcasted_iota(jnp.int32, (_R, _N), 0)
    col_ids = lax.broadcasted_iota(jnp.int32, (_R, _N), 1)
    md = jnp.where(row_ids == col_ids, jnp.inf, d)

    t_row = t_row_ref[0]
    t_col = t_all_ref[0].T
    tmax = jnp.maximum(t_row, t_col)
    adj = (md <= tmax) & jnp.isfinite(md)

    sigma = sigma_ref[0, 0]
    gk = jnp.exp(jnp.where(jnp.isfinite(d), d, 0.0) * (-0.5 / (sigma * sigma)))

    fa = feats_all_ref[0]
    fr = feats_row_ref[0]
    na = jnp.sqrt(jnp.sum(fa * fa, axis=1, keepdims=True))
    nr = jnp.sqrt(jnp.sum(fr * fr, axis=1, keepdims=True))
    fa = fa / jnp.maximum(na, 1e-6)
    fr = fr / jnp.maximum(nr, 1e-6)
    cos = lax.dot_general(fr, fa, (((1,), (1,)), ((), ())),
                          preferred_element_type=jnp.float32)
    aff = jnp.clip((cos + 1.0) * 0.5, 0.0, 1.0)

    wu = jnp.where(adj, gk * aff, 0.0)
    rs = jnp.sum(wu, axis=1, keepdims=True)
    w_ref[0] = (wu / jnp.maximum(rs, 1e-6)).astype(jnp.bfloat16)


def _crf_kernel(w_ref, logits_ref, smooth_ref, out_ref):
    lg = logits_ref[0]
    wb = w_ref[0]
    smooth = smooth_ref[0, 0]
    ii = lax.broadcasted_iota(jnp.int32, (_C, _C), 0)
    jj = lax.broadcasted_iota(jnp.int32, (_C, _C), 1)
    compat = ((ii - jj) ** 2).astype(jnp.float32)
    compat = compat / jnp.maximum(jnp.max(compat), 1.0)

    q = jax.nn.softmax(lg, axis=-1)

    def body(_, carry):
        q, _refined = carry
        qn = jnp.dot(wb, q.astype(jnp.bfloat16),
                     preferred_element_type=jnp.float32)
        pt = jnp.dot(qn, compat, preferred_element_type=jnp.float32)
        refined = lg - smooth * pt
        q = jax.nn.softmax(refined, axis=-1)
        return q, refined

    _, refined = lax.fori_loop(0, _N_ITER, body, (q, lg))
    out_ref[0] = refined


def kernel(logits, rois, appearance_features, raw_sigma, raw_smoothness):
    global _RPW
    B, N, C = logits.shape
    _RPW = (B * N) // _NSUB

    sigma = jnp.maximum(jax.nn.softplus(raw_sigma), 1e-6).astype(jnp.float32)
    smooth = jax.nn.softplus(raw_smoothness).astype(jnp.float32)
    sigma = sigma.reshape(1, 1)
    smooth = smooth.reshape(1, 1)

    soa = pl.pallas_call(
        _soa_kernel,
        grid=(B,),
        in_specs=[pl.BlockSpec((1, N, 6), lambda b: (b, 0, 0))],
        out_specs=pl.BlockSpec((1, 8, N), lambda b: (b, 0, 0)),
        out_shape=jax.ShapeDtypeStruct((B, 8, N), jnp.float32),
    )(rois)

    mesh = plsc.VectorSubcoreMesh(core_axis_name="c", subcore_axis_name="s",
                                  num_cores=2)
    t2d = pl.kernel(
        _sc_thresh_body,
        out_type=jax.ShapeDtypeStruct((B, N), jnp.float32),
        mesh=mesh,
        scratch_types=[
            pltpu.VMEM((8, N), jnp.float32),
            pltpu.VMEM((_RPW,), jnp.float32),
        ],
    )(soa)
    t = t2d.reshape(B, N, 1)

    w = pl.pallas_call(
        _weights_kernel,
        grid=(B, N // _R),
        in_specs=[
            pl.BlockSpec((1, N, 6), lambda b, i: (b, 0, 0)),
            pl.BlockSpec((1, _R, 6), lambda b, i: (b, i, 0)),
            pl.BlockSpec((1, N, _D), lambda b, i: (b, 0, 0)),
            pl.BlockSpec((1, _R, _D), lambda b, i: (b, i, 0)),
            pl.BlockSpec((1, N, 1), lambda b, i: (b, 0, 0)),
            pl.BlockSpec((1, _R, 1), lambda b, i: (b, i, 0)),
            pl.BlockSpec((1, 1), lambda b, i: (0, 0)),
        ],
        out_specs=pl.BlockSpec((1, _R, N), lambda b, i: (b, i, 0)),
        out_shape=jax.ShapeDtypeStruct((B, N, N), jnp.bfloat16),
    )(rois, rois, appearance_features, appearance_features, t, t, sigma)

    refined = pl.pallas_call(
        _crf_kernel,
        grid=(B,),
        in_specs=[
            pl.BlockSpec((1, N, N), lambda b: (b, 0, 0)),
            pl.BlockSpec((1, N, C), lambda b: (b, 0, 0)),
            pl.BlockSpec((1, 1), lambda b: (0, 0)),
        ],
        out_specs=pl.BlockSpec((1, N, C), lambda b: (b, 0, 0)),
        out_shape=jax.ShapeDtypeStruct((B, N, C), jnp.float32),
    )(w, logits, smooth)

    return refined


# speedup vs baseline: 11.2531x; 1.0038x over previous
import functools

import jax
import jax.numpy as jnp
from jax import lax
from jax.experimental import pallas as pl
from jax.experimental.pallas import tpu as pltpu
from jax.experimental.pallas import tpu_sc as plsc

_N = 2048
_C = 32
_D = 256
_KNN = 32
_R = 256
_N_ITER = 5
_NSUB = 32
_RPW = None


def _soa_kernel(rois_ref, soa_ref):
    r = rois_ref[0]
    rt = r.T
    cx = (rt[0:1] + rt[3:4]) * 0.5
    cy = (rt[1:2] + rt[4:5]) * 0.5
    cz = (rt[2:3] + rt[5:6]) * 0.5
    sq = (cx * cx + cy * cy) + cz * cz
    absum = (jnp.abs(rt[0:1]) + jnp.abs(rt[1:2]) + jnp.abs(rt[2:3])
             + jnp.abs(rt[3:4]) + jnp.abs(rt[4:5]) + jnp.abs(rt[5:6]))
    validf = (absum > 0.0).astype(jnp.float32)
    sqp = jnp.where(validf > 0.5, sq, jnp.inf)
    pad = jnp.zeros((4, _N), jnp.float32)
    soa_ref[0] = jnp.concatenate([cx, cy, cz, sqp, pad], axis=0)


def _lane_iota():
    return lax.broadcasted_iota(jnp.int32, (16,), 0)


def _bcast_lane(v, lane):
    idx = jnp.full((16,), lane, jnp.int32)
    return v.at[idx].get(mode="promise_in_bounds")


def _rgather(v, idx):
    return v.at[idx].get(mode="promise_in_bounds")


def _bitonic_sort_vregs(vs):
    li = _lane_iota()
    n = len(vs) * 16
    k = 2
    while k <= n:
        s = k // 2
        while s >= 1:
            if s >= 16:
                sv = s // 16
                for p in range(len(vs) // 2):
                    v = ((p & ~(sv - 1)) << 1) | (p & (sv - 1))
                    up = ((16 * v) & k) == 0
                    a, b = vs[v], vs[v + sv]
                    lo = jnp.minimum(a, b)
                    hi = jnp.maximum(a, b)
                    vs[v], vs[v + sv] = (lo, hi) if up else (hi, lo)
            else:
                ls = s.bit_length() - 1
                lk = k.bit_length() - 1
                for v in range(len(vs)):
                    a = vs[v]
                    b = _rgather(a, li ^ s)
                    if k >= 16:
                        if ((16 * v) & k) != 0:
                            tm = (li & s) != 0
                        else:
                            tm = (li & s) == 0
                    else:
                        tm = (((li >> ls) ^ (li >> lk)) & 1) == 0
                    vs[v] = jnp.where(tm, jnp.minimum(a, b), jnp.maximum(a, b))
            s //= 2
        k *= 2
    return vs


def _sc_thresh_body(soa_hbm, t_hbm, soa_v, tbuf):
    nc = 2
    wid = lax.axis_index("s") * nc + lax.axis_index("c")
    segs_per_batch = _N // _RPW
    b = wid // segs_per_batch
    seg = wid % segs_per_batch
    row0 = seg * _RPW

    pltpu.sync_copy(soa_hbm.at[b], soa_v)

    inf16 = jnp.full((16,), jnp.inf, jnp.float32)
    li = _lane_iota()
    nvec = _N // 16

    def row_body(j, acc):
        nrow = row0 + j
        base = (nrow // 16) * 16
        lane = nrow % 16
        xr = _bcast_lane(soa_v[0, pl.ds(base, 16)], lane)
        yr = _bcast_lane(soa_v[1, pl.ds(base, 16)], lane)
        zr = _bcast_lane(soa_v[2, pl.ds(base, 16)], lane)
        sqr = _bcast_lane(soa_v[3, pl.ds(base, 16)], lane)

        def stream_body(jv, ms):
            o = jv * 16
            vx = soa_v[0, pl.ds(o, 16)]
            vy = soa_v[1, pl.ds(o, 16)]
            vz = soa_v[2, pl.ds(o, 16)]
            vs = soa_v[3, pl.ds(o, 16)]
            dot = (vx * xr + vy * yr) + vz * zr
            c = jnp.maximum((sqr - (dot + dot)) + vs, 0.0)
            out = []
            for i in range(8):
                lo = jnp.minimum(ms[i], c)
                c = jnp.maximum(ms[i], c)
                out.append(lo)
            return tuple(out)

        ms = lax.fori_loop(0, nvec, stream_body, (inf16,) * 8)

        vs_ = _bitonic_sort_vregs(list(ms))
        t = _bcast_lane(vs_[2], 0)

        acc = jnp.where(li == (j & 15), t, acc)
        tbuf[pl.ds((j // 16) * 16, 16)] = acc
        return acc

    lax.fori_loop(0, _RPW, row_body, jnp.zeros((16,), jnp.float32))
    pltpu.sync_copy(tbuf, t_hbm.at[b, pl.ds(row0, _RPW)])


def _weights_kernel(rois_all_ref, rois_row_ref, feats_all_ref, feats_row_ref,
                    t_all_ref, t_row_ref, sigma_ref, w_ref):
    i = pl.program_id(1)
    ra = rois_all_ref[0]
    rr = rois_row_ref[0]

    cxr = (rr[:, 0:1] + rr[:, 3:4]) * 0.5
    cyr = (rr[:, 1:2] + rr[:, 4:5]) * 0.5
    czr = (rr[:, 2:3] + rr[:, 5:6]) * 0.5
    sq_r = (cxr * cxr + cyr * cyr) + czr * czr
    absum_r = (jnp.abs(rr[:, 0:1]) + jnp.abs(rr[:, 1:2]) + jnp.abs(rr[:, 2:3])
               + jnp.abs(rr[:, 3:4]) + jnp.abs(rr[:, 4:5]) + jnp.abs(rr[:, 5:6]))
    sqp_r = jnp.where(absum_r > 0.0, sq_r, jnp.inf)

    rat = ra.T
    cxa = (rat[0:1] + rat[3:4]) * 0.5
    cya = (rat[1:2] + rat[4:5]) * 0.5
    cza = (rat[2:3] + rat[5:6]) * 0.5
    sq_a = (cxa * cxa + cya * cya) + cza * cza
    absum_a = (jnp.abs(rat[0:1]) + jnp.abs(rat[1:2]) + jnp.abs(rat[2:3])
               + jnp.abs(rat[3:4]) + jnp.abs(rat[4:5]) + jnp.abs(rat[5:6]))
    validf_a = (absum_a > 0.0).astype(jnp.float32)
    sqp_a = jnp.where(validf_a > 0.5, sq_a, jnp.inf)

    dot = (cxa * cxr + cya * cyr) + cza * czr
    d = jnp.maximum((sqp_r - (dot + dot)) + sqp_a, 0.0)

    row_ids = i * _R + lax.broadcasted_iota(jnp.int32, (_R, _N), 0)
    col_ids = lax.broadcasted_iota(jnp.int32, (_R, _N), 1)
    md = jnp.where(row_ids == col_ids, jnp.inf, d)

    t_row = t_row_ref[0]
    t_col = t_all_ref[0].T
    tmax = jnp.maximum(t_row, t_col)
    adj = (md <= tmax) & jnp.isfinite(md)

    sigma = sigma_ref[0, 0]
    gk = jnp.exp(jnp.where(jnp.isfinite(d), d, 0.0) * (-0.5 / (sigma * sigma)))

    fa = feats_all_ref[0]
    fr = feats_row_ref[0]
    na = jnp.sqrt(jnp.sum(fa * fa, axis=1, keepdims=True))
    nr = jnp.sqrt(jnp.sum(fr * fr, axis=1, keepdims=True))
    fa = fa / jnp.maximum(na, 1e-6)
    fr = fr / jnp.maximum(nr, 1e-6)
    cos = lax.dot_general(fr.astype(jnp.bfloat16), fa.astype(jnp.bfloat16),
                          (((1,), (1,)), ((), ())),
                          preferred_element_type=jnp.float32)
    aff = jnp.clip((cos + 1.0) * 0.5, 0.0, 1.0)

    wu = jnp.where(adj, gk * aff, 0.0)
    rs = jnp.sum(wu, axis=1, keepdims=True)
    w_ref[0] = (wu / jnp.maximum(rs, 1e-6)).astype(jnp.bfloat16)


def _crf_kernel(w_ref, logits_ref, smooth_ref, out_ref):
    lg = logits_ref[0]
    wb = w_ref[0]
    smooth = smooth_ref[0, 0]
    ii = lax.broadcasted_iota(jnp.int32, (_C, _C), 0)
    jj = lax.broadcasted_iota(jnp.int32, (_C, _C), 1)
    compat = ((ii - jj) ** 2).astype(jnp.float32)
    compat = compat / jnp.maximum(jnp.max(compat), 1.0)

    q = jax.nn.softmax(lg, axis=-1)

    def body(_, carry):
        q, _refined = carry
        qn = jnp.dot(wb, q.astype(jnp.bfloat16),
                     preferred_element_type=jnp.float32)
        pt = jnp.dot(qn, compat, preferred_element_type=jnp.float32)
        refined = lg - smooth * pt
        q = jax.nn.softmax(refined, axis=-1)
        return q, refined

    _, refined = lax.fori_loop(0, _N_ITER, body, (q, lg))
    out_ref[0] = refined


def kernel(logits, rois, appearance_features, raw_sigma, raw_smoothness):
    global _RPW
    B, N, C = logits.shape
    _RPW = (B * N) // _NSUB

    sigma = jnp.maximum(jax.nn.softplus(raw_sigma), 1e-6).astype(jnp.float32)
    smooth = jax.nn.softplus(raw_smoothness).astype(jnp.float32)
    sigma = sigma.reshape(1, 1)
    smooth = smooth.reshape(1, 1)

    soa = pl.pallas_call(
        _soa_kernel,
        grid=(B,),
        in_specs=[pl.BlockSpec((1, N, 6), lambda b: (b, 0, 0))],
        out_specs=pl.BlockSpec((1, 8, N), lambda b: (b, 0, 0)),
        out_shape=jax.ShapeDtypeStruct((B, 8, N), jnp.float32),
    )(rois)

    mesh = plsc.VectorSubcoreMesh(core_axis_name="c", subcore_axis_name="s",
                                  num_cores=2)
    t2d = pl.kernel(
        _sc_thresh_body,
        out_type=jax.ShapeDtypeStruct((B, N), jnp.float32),
        mesh=mesh,
        scratch_types=[
            pltpu.VMEM((8, N), jnp.float32),
            pltpu.VMEM((_RPW,), jnp.float32),
        ],
    )(soa)
    t = t2d.reshape(B, N, 1)

    w = pl.pallas_call(
        _weights_kernel,
        grid=(B, N // _R),
        in_specs=[
            pl.BlockSpec((1, N, 6), lambda b, i: (b, 0, 0)),
            pl.BlockSpec((1, _R, 6), lambda b, i: (b, i, 0)),
            pl.BlockSpec((1, N, _D), lambda b, i: (b, 0, 0)),
            pl.BlockSpec((1, _R, _D), lambda b, i: (b, i, 0)),
            pl.BlockSpec((1, N, 1), lambda b, i: (b, 0, 0)),
            pl.BlockSpec((1, _R, 1), lambda b, i: (b, i, 0)),
            pl.BlockSpec((1, 1), lambda b, i: (0, 0)),
        ],
        out_specs=pl.BlockSpec((1, _R, N), lambda b, i: (b, i, 0)),
        out_shape=jax.ShapeDtypeStruct((B, N, N), jnp.bfloat16),
    )(rois, rois, appearance_features, appearance_features, t, t, sigma)

    refined = pl.pallas_call(
        _crf_kernel,
        grid=(B,),
        in_specs=[
            pl.BlockSpec((1, N, N), lambda b: (b, 0, 0)),
            pl.BlockSpec((1, N, C), lambda b: (b, 0, 0)),
            pl.BlockSpec((1, 1), lambda b: (0, 0)),
        ],
        out_specs=pl.BlockSpec((1, N, C), lambda b: (b, 0, 0)),
        out_shape=jax.ShapeDtypeStruct((B, N, C), jnp.float32),
    )(w, logits, smooth)

    return refined


# speedup vs baseline: 12.3753x; 1.0997x over previous
import functools

import jax
import jax.numpy as jnp
from jax import lax
from jax.experimental import pallas as pl
from jax.experimental.pallas import tpu as pltpu
from jax.experimental.pallas import tpu_sc as plsc

_N = 2048
_C = 32
_D = 256
_KNN = 32
_R = 256
_N_ITER = 5
_NSUB = 32
_RPW = None


def _soa_kernel(rois_ref, soa_ref):
    r = rois_ref[0]
    rt = r.T
    cx = (rt[0:1] + rt[3:4]) * 0.5
    cy = (rt[1:2] + rt[4:5]) * 0.5
    cz = (rt[2:3] + rt[5:6]) * 0.5
    sq = (cx * cx + cy * cy) + cz * cz
    absum = (jnp.abs(rt[0:1]) + jnp.abs(rt[1:2]) + jnp.abs(rt[2:3])
             + jnp.abs(rt[3:4]) + jnp.abs(rt[4:5]) + jnp.abs(rt[5:6]))
    validf = (absum > 0.0).astype(jnp.float32)
    sqp = jnp.where(validf > 0.5, sq, jnp.inf)
    pad = jnp.zeros((4, _N), jnp.float32)
    soa_ref[0] = jnp.concatenate([cx, cy, cz, sqp, pad], axis=0)


def _lane_iota():
    return lax.broadcasted_iota(jnp.int32, (16,), 0)


def _bcast_lane(v, lane):
    idx = jnp.full((16,), lane, jnp.int32)
    return v.at[idx].get(mode="promise_in_bounds")


def _rgather(v, idx):
    return v.at[idx].get(mode="promise_in_bounds")


def _bitonic_sort_vregs(vs):
    li = _lane_iota()
    n = len(vs) * 16
    k = 2
    while k <= n:
        s = k // 2
        while s >= 1:
            if s >= 16:
                sv = s // 16
                for p in range(len(vs) // 2):
                    v = ((p & ~(sv - 1)) << 1) | (p & (sv - 1))
                    up = ((16 * v) & k) == 0
                    a, b = vs[v], vs[v + sv]
                    lo = jnp.minimum(a, b)
                    hi = jnp.maximum(a, b)
                    vs[v], vs[v + sv] = (lo, hi) if up else (hi, lo)
            else:
                ls = s.bit_length() - 1
                lk = k.bit_length() - 1
                for v in range(len(vs)):
                    a = vs[v]
                    b = _rgather(a, li ^ s)
                    if k >= 16:
                        if ((16 * v) & k) != 0:
                            tm = (li & s) != 0
                        else:
                            tm = (li & s) == 0
                    else:
                        tm = (((li >> ls) ^ (li >> lk)) & 1) == 0
                    vs[v] = jnp.where(tm, jnp.minimum(a, b), jnp.maximum(a, b))
            s //= 2
        k *= 2
    return vs


def _sc_thresh_body(soa_hbm, t_hbm, soa_v, tbuf):
    nc = 2
    wid = lax.axis_index("s") * nc + lax.axis_index("c")
    segs_per_batch = _N // _RPW
    b = wid // segs_per_batch
    seg = wid % segs_per_batch
    row0 = seg * _RPW

    pltpu.sync_copy(soa_hbm.at[b], soa_v)

    inf16 = jnp.full((16,), jnp.inf, jnp.float32)
    li = _lane_iota()
    nvec = _N // 16

    def row_body(j, acc):
        nrow = row0 + j
        base = (nrow // 16) * 16
        lane = nrow % 16
        xr = _bcast_lane(soa_v[0, pl.ds(base, 16)], lane)
        yr = _bcast_lane(soa_v[1, pl.ds(base, 16)], lane)
        zr = _bcast_lane(soa_v[2, pl.ds(base, 16)], lane)
        sqr = _bcast_lane(soa_v[3, pl.ds(base, 16)], lane)

        def stream_body(jv, ms):
            o = jv * 16
            vx = soa_v[0, pl.ds(o, 16)]
            vy = soa_v[1, pl.ds(o, 16)]
            vz = soa_v[2, pl.ds(o, 16)]
            vs = soa_v[3, pl.ds(o, 16)]
            dot = (vx * xr + vy * yr) + vz * zr
            c = jnp.maximum((sqr - (dot + dot)) + vs, 0.0)
            out = []
            for i in range(8):
                lo = jnp.minimum(ms[i], c)
                c = jnp.maximum(ms[i], c)
                out.append(lo)
            return tuple(out)

        ms = lax.fori_loop(0, nvec, stream_body, (inf16,) * 8)

        vs_ = _bitonic_sort_vregs(list(ms))
        t = _bcast_lane(vs_[2], 0)

        acc = jnp.where(li == (j & 15), t, acc)
        tbuf[pl.ds((j // 16) * 16, 16)] = acc
        return acc

    lax.fori_loop(0, _RPW, row_body, jnp.zeros((16,), jnp.float32))
    pltpu.sync_copy(tbuf, t_hbm.at[b, pl.ds(row0, _RPW)])


def _rowcol_geom(ra, rr):
    cxr = (rr[:, 0:1] + rr[:, 3:4]) * 0.5
    cyr = (rr[:, 1:2] + rr[:, 4:5]) * 0.5
    czr = (rr[:, 2:3] + rr[:, 5:6]) * 0.5
    sq_r = (cxr * cxr + cyr * cyr) + czr * czr
    absum_r = (jnp.abs(rr[:, 0:1]) + jnp.abs(rr[:, 1:2]) + jnp.abs(rr[:, 2:3])
               + jnp.abs(rr[:, 3:4]) + jnp.abs(rr[:, 4:5]) + jnp.abs(rr[:, 5:6]))
    validf_r = (absum_r > 0.0).astype(jnp.float32)
    sqp_r = jnp.where(validf_r > 0.5, sq_r, jnp.inf)

    rat = ra.T
    cxa = (rat[0:1] + rat[3:4]) * 0.5
    cya = (rat[1:2] + rat[4:5]) * 0.5
    cza = (rat[2:3] + rat[5:6]) * 0.5
    sq_a = (cxa * cxa + cya * cya) + cza * cza
    absum_a = (jnp.abs(rat[0:1]) + jnp.abs(rat[1:2]) + jnp.abs(rat[2:3])
               + jnp.abs(rat[3:4]) + jnp.abs(rat[4:5]) + jnp.abs(rat[5:6]))
    validf_a = (absum_a > 0.0).astype(jnp.float32)
    sqp_a = jnp.where(validf_a > 0.5, sq_a, jnp.inf)

    dot = (cxa * cxr + cya * cyr) + cza * czr
    d = jnp.maximum((sqp_r - (dot + dot)) + sqp_a, 0.0)
    return d


def _aff_kernel(rois_all_ref, rois_row_ref, feats_all_ref, feats_row_ref,
                sigma_ref, aff_ref):
    d = _rowcol_geom(rois_all_ref[0], rois_row_ref[0])

    sigma = sigma_ref[0, 0]
    gk = jnp.exp(jnp.where(jnp.isfinite(d), d, 0.0) * (-0.5 / (sigma * sigma)))

    fa = feats_all_ref[0]
    fr = feats_row_ref[0]
    na = jnp.sqrt(jnp.sum(fa * fa, axis=1, keepdims=True))
    nr = jnp.sqrt(jnp.sum(fr * fr, axis=1, keepdims=True))
    fa = fa / jnp.maximum(na, 1e-6)
    fr = fr / jnp.maximum(nr, 1e-6)
    cos = lax.dot_general(fr.astype(jnp.bfloat16), fa.astype(jnp.bfloat16),
                          (((1,), (1,)), ((), ())),
                          preferred_element_type=jnp.float32)
    aff = jnp.clip((cos + 1.0) * 0.5, 0.0, 1.0)
    aff_ref[0] = (gk * aff).astype(jnp.bfloat16)


def _weights_kernel(rois_all_ref, rois_row_ref, aff_ref,
                    t_all_ref, t_row_ref, w_ref):
    i = pl.program_id(1)
    md0 = _rowcol_geom(rois_all_ref[0], rois_row_ref[0])
    row_ids = i * _R + lax.broadcasted_iota(jnp.int32, (_R, _N), 0)
    col_ids = lax.broadcasted_iota(jnp.int32, (_R, _N), 1)
    md = jnp.where(row_ids == col_ids, jnp.inf, md0)

    t_row = t_row_ref[0]
    t_col = t_all_ref[0].T
    tmax = jnp.maximum(t_row, t_col)
    adj = (md <= tmax) & jnp.isfinite(md)

    wu = jnp.where(adj, aff_ref[0].astype(jnp.float32), 0.0)
    rs = jnp.sum(wu, axis=1, keepdims=True)
    w_ref[0] = (wu / jnp.maximum(rs, 1e-6)).astype(jnp.bfloat16)


def _crf_kernel(w_ref, logits_ref, smooth_ref, out_ref):
    lg = logits_ref[0]
    wb = w_ref[0]
    smooth = smooth_ref[0, 0]
    ii = lax.broadcasted_iota(jnp.int32, (_C, _C), 0)
    jj = lax.broadcasted_iota(jnp.int32, (_C, _C), 1)
    compat = ((ii - jj) ** 2).astype(jnp.float32)
    compat = compat / jnp.maximum(jnp.max(compat), 1.0)

    q = jax.nn.softmax(lg, axis=-1)

    def body(_, carry):
        q, _refined = carry
        qn = jnp.dot(wb, q.astype(jnp.bfloat16),
                     preferred_element_type=jnp.float32)
        pt = jnp.dot(qn, compat, preferred_element_type=jnp.float32)
        refined = lg - smooth * pt
        q = jax.nn.softmax(refined, axis=-1)
        return q, refined

    _, refined = lax.fori_loop(0, _N_ITER, body, (q, lg))
    out_ref[0] = refined


def kernel(logits, rois, appearance_features, raw_sigma, raw_smoothness):
    global _RPW
    B, N, C = logits.shape
    _RPW = (B * N) // _NSUB

    sigma = jnp.maximum(jax.nn.softplus(raw_sigma), 1e-6).astype(jnp.float32)
    smooth = jax.nn.softplus(raw_smoothness).astype(jnp.float32)
    sigma = sigma.reshape(1, 1)
    smooth = smooth.reshape(1, 1)

    soa = pl.pallas_call(
        _soa_kernel,
        grid=(B,),
        in_specs=[pl.BlockSpec((1, N, 6), lambda b: (b, 0, 0))],
        out_specs=pl.BlockSpec((1, 8, N), lambda b: (b, 0, 0)),
        out_shape=jax.ShapeDtypeStruct((B, 8, N), jnp.float32),
    )(rois)

    mesh = plsc.VectorSubcoreMesh(core_axis_name="c", subcore_axis_name="s",
                                  num_cores=2)
    t2d = pl.kernel(
        _sc_thresh_body,
        out_type=jax.ShapeDtypeStruct((B, N), jnp.float32),
        mesh=mesh,
        scratch_types=[
            pltpu.VMEM((8, N), jnp.float32),
            pltpu.VMEM((_RPW,), jnp.float32),
        ],
    )(soa)
    t = t2d.reshape(B, N, 1)

    aff = pl.pallas_call(
        _aff_kernel,
        grid=(B, N // _R),
        in_specs=[
            pl.BlockSpec((1, N, 6), lambda b, i: (b, 0, 0)),
            pl.BlockSpec((1, _R, 6), lambda b, i: (b, i, 0)),
            pl.BlockSpec((1, N, _D), lambda b, i: (b, 0, 0)),
            pl.BlockSpec((1, _R, _D), lambda b, i: (b, i, 0)),
            pl.BlockSpec((1, 1), lambda b, i: (0, 0)),
        ],
        out_specs=pl.BlockSpec((1, _R, N), lambda b, i: (b, i, 0)),
        out_shape=jax.ShapeDtypeStruct((B, N, N), jnp.bfloat16),
    )(rois, rois, appearance_features, appearance_features, sigma)

    w = pl.pallas_call(
        _weights_kernel,
        grid=(B, N // _R),
        in_specs=[
            pl.BlockSpec((1, N, 6), lambda b, i: (b, 0, 0)),
            pl.BlockSpec((1, _R, 6), lambda b, i: (b, i, 0)),
            pl.BlockSpec((1, _R, N), lambda b, i: (b, i, 0)),
            pl.BlockSpec((1, N, 1), lambda b, i: (b, 0, 0)),
            pl.BlockSpec((1, _R, 1), lambda b, i: (b, i, 0)),
        ],
        out_specs=pl.BlockSpec((1, _R, N), lambda b, i: (b, i, 0)),
        out_shape=jax.ShapeDtypeStruct((B, N, N), jnp.bfloat16),
    )(rois, rois, aff, t, t)

    refined = pl.pallas_call(
        _crf_kernel,
        grid=(B,),
        in_specs=[
            pl.BlockSpec((1, N, N), lambda b: (b, 0, 0)),
            pl.BlockSpec((1, N, C), lambda b: (b, 0, 0)),
            pl.BlockSpec((1, 1), lambda b: (0, 0)),
        ],
        out_specs=pl.BlockSpec((1, N, C), lambda b: (b, 0, 0)),
        out_shape=jax.ShapeDtypeStruct((B, N, C), jnp.float32),
    )(w, logits, smooth)

    return refined
